# baseline (speedup 1.0000x reference)
import functools
import math

import jax
import jax.numpy as jnp
from jax import lax
from jax.experimental import pallas as pl
from jax.experimental.pallas import tpu as pltpu

G = 512
H = 8
SW = 8
GP = G + 2 * SW


def _fused_body(nf_ref, seg_ref, watt_ref, wvbig_ref, wo_ref, bo_ref,
                g1_ref, b1_ref, g2_ref, b2_ref, out_ref, acc_ref, accs_ref,
                *, bsz, nb, d):
    i = pl.program_id(0)

    @pl.when(i == 0)
    def _init():
        acc_ref[...] = jnp.zeros_like(acc_ref)
        accs_ref[...] = jnp.zeros_like(accs_ref)

    nfb = nf_ref[...]
    ltt = lax.dot_general(watt_ref[...], nfb, (((0,), (1,)), ((), ())),
                          preferred_element_type=jnp.float32)
    ett = jnp.exp(ltt)
    colj = lax.broadcasted_iota(jnp.int32, (H, 16 * SW), 1) % 16
    rowj = lax.broadcasted_iota(jnp.int32, (H, 16 * SW), 0)
    tilemat = (colj == rowj).astype(jnp.float32)
    const1 = (lax.broadcasted_iota(jnp.int32, (1, 16 * SW), 1) % 16
              == H).astype(jnp.float32)
    wtile = lax.dot_general(ett, tilemat, (((0,), (0,)), ((), ())),
                            preferred_element_type=jnp.float32) + const1
    onesb = jnp.ones((bsz, 1), jnp.float32)
    segb2 = seg_ref[0, 0, :][:, None]
    colgrp = lax.broadcasted_iota(jnp.int32, (1, 16 * SW), 1) // 16
    lo = seg_ref[0, 0, 0]
    hi = seg_ref[0, 0, bsz - 1]
    nwin = (hi - lo) // SW + 1

    def win_body(jw, carry):
        base = lo + jw * SW
        match = (segb2 == base + colgrp)
        ew = jnp.where(match, wtile, 0.0)
        contrib = lax.dot_general(ew, nfb, (((0,), (0,)), ((), ())),
                                  preferred_element_type=jnp.float32)
        acc_ref[pl.ds(base, SW), :, :] += contrib.reshape(SW, 16, d)
        scon = lax.dot_general(ew, onesb, (((0,), (0,)), ((), ())),
                               preferred_element_type=jnp.float32)
        accs_ref[pl.ds(base, SW), :, :] += scon.reshape(SW, 1, 16)
        return carry

    lax.fori_loop(0, nwin, win_body, 0)

    @pl.when(i == nb - 1)
    def _epilogue():
        x0 = acc_ref[:G, H, :]
        sall = accs_ref[:G, 0, :]
        cinv = 1.0 / jnp.maximum(sall[:, H:H + 1], 1.0)
        sinv = cinv / (sall[:, :H] + 1e-16)
        rep = (lax.broadcasted_iota(jnp.int32, (H, d), 1) // (d // H)
               == lax.broadcasted_iota(jnp.int32, (H, d), 0)
               ).astype(jnp.float32)
        scale = lax.dot_general(sinv, rep, (((1,), (0,)), ((), ())),
                                preferred_element_type=jnp.float32)
        ybig = jnp.concatenate([acc_ref[:G, h, :] for h in range(H)],
                               axis=1)
        x = jnp.dot(ybig, wvbig_ref[...],
                    preferred_element_type=jnp.float32) * scale

        def ln(v, g, b):
            mu = jnp.mean(v, axis=1, keepdims=True)
            var = jnp.mean(jnp.square(v - mu), axis=1, keepdims=True)
            return g * (v - mu) / jnp.sqrt(var + 1e-3) + b

        x = ln(x, g1_ref[...], b1_ref[...])
        x = jnp.maximum(jnp.dot(x, wo_ref[...],
                                preferred_element_type=jnp.float32)
                        + bo_ref[...], 0.0)
        x = ln(x, g2_ref[...], b2_ref[...])
        out_ref[...] = x + x0


def kernel(nf, nId, vnt, WQ, WK, WV, WO, bO, g1, b1, g2, b2):
    n, d = nf.shape
    dk = d // H
    seg = nId.astype(jnp.int32)

    q = (vnt @ WQ).reshape(H, dk)
    watt = (WK.reshape(d, H, dk) * q[None, :, :]).sum(-1) / math.sqrt(dk)
    wvr = WV.reshape(d, H, dk).transpose(1, 0, 2)
    wvbig = (jnp.eye(H, dtype=jnp.float32)[:, None, :, None]
             * wvr[:, :, None, :]).reshape(H * d, d)

    bsz = 1000
    assert n % bsz == 0
    nb = n // bsz
    seg3 = seg.reshape(nb, 1, bsz)

    out = pl.pallas_call(
        functools.partial(_fused_body, bsz=bsz, nb=nb, d=d),
        grid=(nb,),
        in_specs=[
            pl.BlockSpec((bsz, d), lambda i: (i, 0)),
            pl.BlockSpec((1, 1, bsz), lambda i: (i, 0, 0)),
            pl.BlockSpec((d, H), lambda i: (0, 0)),
            pl.BlockSpec((H * d, d), lambda i: (0, 0)),
            pl.BlockSpec((d, d), lambda i: (0, 0)),
            pl.BlockSpec((1, d), lambda i: (0, 0)),
            pl.BlockSpec((1, d), lambda i: (0, 0)),
            pl.BlockSpec((1, d), lambda i: (0, 0)),
            pl.BlockSpec((1, d), lambda i: (0, 0)),
            pl.BlockSpec((1, d), lambda i: (0, 0)),
        ],
        out_specs=pl.BlockSpec((G, d), lambda i: (0, 0)),
        out_shape=jax.ShapeDtypeStruct((G, d), jnp.float32),
        scratch_shapes=[pltpu.VMEM((GP, 16, d), jnp.float32),
                        pltpu.VMEM((GP, 1, 16), jnp.float32)],
    )(nf, seg3, watt, wvbig, WO, bO.reshape(1, d), g1.reshape(1, d),
      b1.reshape(1, d), g2.reshape(1, d), b2.reshape(1, d))
    return out


# speedup vs baseline: 1.1104x; 1.1104x over previous
import functools
import math

import jax
import jax.numpy as jnp
from jax import lax
from jax.experimental import pallas as pl
from jax.experimental.pallas import tpu as pltpu

G = 512
H = 8
SW = 8
GP = G + 2 * SW
AW = 384


def _fused_body(nf_ref, seg_ref, watt_ref, wvbig_ref, wo_ref, bo_ref,
                g1_ref, b1_ref, g2_ref, b2_ref, out_ref, acc_ref, *,
                bsz, nb, d):
    i = pl.program_id(0)

    @pl.when(i == 0)
    def _init():
        acc_ref[...] = jnp.zeros_like(acc_ref)

    nfb = nf_ref[...]
    aug = jnp.concatenate(
        [nfb, jnp.ones((bsz, 1), jnp.float32),
         jnp.zeros((bsz, AW - d - 1), jnp.float32)], axis=1)
    ltt = lax.dot_general(watt_ref[...], nfb, (((0,), (1,)), ((), ())),
                          preferred_element_type=jnp.float32)
    ett = jnp.exp(ltt)
    colj = lax.broadcasted_iota(jnp.int32, (H, 16 * SW), 1) % 16
    rowj = lax.broadcasted_iota(jnp.int32, (H, 16 * SW), 0)
    tilemat = (colj == rowj).astype(jnp.float32)
    const1 = (lax.broadcasted_iota(jnp.int32, (1, 16 * SW), 1) % 16
              == H).astype(jnp.float32)
    wtile = lax.dot_general(ett, tilemat, (((0,), (0,)), ((), ())),
                            preferred_element_type=jnp.float32) + const1
    segb2 = seg_ref[0, 0, :][:, None]
    colgrp = lax.broadcasted_iota(jnp.int32, (1, 16 * SW), 1) // 16
    lo = seg_ref[0, 0, 0]
    hi = seg_ref[0, 0, bsz - 1]
    nwin = (hi - lo) // SW + 1

    def win_body(jw, carry):
        base = lo + jw * SW
        match = (segb2 == base + colgrp)
        ew = jnp.where(match, wtile, 0.0)
        contrib = lax.dot_general(ew, aug, (((0,), (0,)), ((), ())),
                                  preferred_element_type=jnp.float32)
        acc_ref[pl.ds(base, SW), :, :] += contrib.reshape(SW, 16, AW)
        return carry

    lax.fori_loop(0, nwin, win_body, 0)

    @pl.when(i == nb - 1)
    def _epilogue():
        planes = [acc_ref[:G, h, :] for h in range(H + 1)]
        x0 = planes[H][:, :d]
        cinv = 1.0 / jnp.maximum(planes[H][:, d:d + 1], 1.0)
        sinv = jnp.concatenate([planes[h][:, d:d + 1] for h in range(H)],
                               axis=1)
        sinv = cinv / (sinv + 1e-16)
        rep = (lax.broadcasted_iota(jnp.int32, (H, d), 1) // (d // H)
               == lax.broadcasted_iota(jnp.int32, (H, d), 0)
               ).astype(jnp.float32)
        scale = lax.dot_general(sinv, rep, (((1,), (0,)), ((), ())),
                                preferred_element_type=jnp.float32)
        ybig = jnp.concatenate([planes[h][:, :d] for h in range(H)],
                               axis=1)
        x = jnp.dot(ybig, wvbig_ref[...],
                    preferred_element_type=jnp.float32) * scale

        def ln(v, g, b):
            mu = jnp.mean(v, axis=1, keepdims=True)
            var = jnp.mean(jnp.square(v - mu), axis=1, keepdims=True)
            return g * (v - mu) / jnp.sqrt(var + 1e-3) + b

        x = ln(x, g1_ref[...], b1_ref[...])
        x = jnp.maximum(jnp.dot(x, wo_ref[...],
                                preferred_element_type=jnp.float32)
                        + bo_ref[...], 0.0)
        x = ln(x, g2_ref[...], b2_ref[...])
        out_ref[...] = x + x0


def kernel(nf, nId, vnt, WQ, WK, WV, WO, bO, g1, b1, g2, b2):
    n, d = nf.shape
    dk = d // H
    seg = nId.astype(jnp.int32)

    q = (vnt @ WQ).reshape(H, dk)
    watt = (WK.reshape(d, H, dk) * q[None, :, :]).sum(-1) / math.sqrt(dk)
    wvr = WV.reshape(d, H, dk).transpose(1, 0, 2)
    wvbig = (jnp.eye(H, dtype=jnp.float32)[:, None, :, None]
             * wvr[:, :, None, :]).reshape(H * d, d)

    bsz = 1000
    assert n % bsz == 0
    nb = n // bsz
    seg3 = seg.reshape(nb, 1, bsz)

    out = pl.pallas_call(
        functools.partial(_fused_body, bsz=bsz, nb=nb, d=d),
        grid=(nb,),
        in_specs=[
            pl.BlockSpec((bsz, d), lambda i: (i, 0)),
            pl.BlockSpec((1, 1, bsz), lambda i: (i, 0, 0)),
            pl.BlockSpec((d, H), lambda i: (0, 0)),
            pl.BlockSpec((H * d, d), lambda i: (0, 0)),
            pl.BlockSpec((d, d), lambda i: (0, 0)),
            pl.BlockSpec((1, d), lambda i: (0, 0)),
            pl.BlockSpec((1, d), lambda i: (0, 0)),
            pl.BlockSpec((1, d), lambda i: (0, 0)),
            pl.BlockSpec((1, d), lambda i: (0, 0)),
            pl.BlockSpec((1, d), lambda i: (0, 0)),
        ],
        out_specs=pl.BlockSpec((G, d), lambda i: (0, 0)),
        out_shape=jax.ShapeDtypeStruct((G, d), jnp.float32),
        scratch_shapes=[pltpu.VMEM((GP, 16, AW), jnp.float32)],
    )(nf, seg3, watt, wvbig, WO, bO.reshape(1, d), g1.reshape(1, d),
      b1.reshape(1, d), g2.reshape(1, d), b2.reshape(1, d))
    return out


# speedup vs baseline: 1.2322x; 1.1097x over previous
import functools
import math

import jax
import jax.numpy as jnp
from jax import lax
from jax.experimental import pallas as pl
from jax.experimental.pallas import tpu as pltpu

G = 512
H = 8
SW = 16
GP = G + 2 * SW
AW = 384


def _fused_body(nf_ref, seg_ref, watt_ref, wvbig_ref, wo_ref, bo_ref,
                g1_ref, b1_ref, g2_ref, b2_ref, out_ref, acc_ref, *,
                bsz, nb, d):
    i = pl.program_id(0)

    @pl.when(i == 0)
    def _init():
        acc_ref[...] = jnp.zeros_like(acc_ref)

    nfb = nf_ref[...]
    aug = jnp.concatenate(
        [nfb, jnp.ones((bsz, 1), jnp.float32),
         jnp.zeros((bsz, AW - d - 1), jnp.float32)], axis=1)
    ltt = lax.dot_general(watt_ref[...], nfb, (((0,), (1,)), ((), ())),
                          preferred_element_type=jnp.float32)
    ett = jnp.exp(ltt)
    colj = lax.broadcasted_iota(jnp.int32, (H, 16 * SW), 1) % 16
    rowj = lax.broadcasted_iota(jnp.int32, (H, 16 * SW), 0)
    tilemat = (colj == rowj).astype(jnp.float32)
    const1 = (lax.broadcasted_iota(jnp.int32, (1, 16 * SW), 1) % 16
              == H).astype(jnp.float32)
    wtile = lax.dot_general(ett, tilemat, (((0,), (0,)), ((), ())),
                            preferred_element_type=jnp.float32) + const1
    segb2 = seg_ref[0, 0, :][:, None]
    colgrp = lax.broadcasted_iota(jnp.int32, (1, 16 * SW), 1) // 16
    lo = seg_ref[0, 0, 0]
    hi = seg_ref[0, 0, bsz - 1]
    nwin = (hi - lo) // SW + 1

    def win_body(jw, carry):
        base = lo + jw * SW
        match = (segb2 == base + colgrp)
        ew = jnp.where(match, wtile, 0.0)
        contrib = lax.dot_general(ew, aug, (((0,), (0,)), ((), ())),
                                  preferred_element_type=jnp.float32)
        acc_ref[pl.ds(base, SW), :, :] += contrib.reshape(SW, 16, AW)
        return carry

    lax.fori_loop(0, nwin, win_body, 0)

    @pl.when(i == nb - 1)
    def _epilogue():
        planes = [acc_ref[:G, h, :] for h in range(H + 1)]
        x0 = planes[H][:, :d]
        cinv = 1.0 / jnp.maximum(planes[H][:, d:d + 1], 1.0)
        sinv = jnp.concatenate([planes[h][:, d:d + 1] for h in range(H)],
                               axis=1)
        sinv = cinv / (sinv + 1e-16)
        rep = (lax.broadcasted_iota(jnp.int32, (H, d), 1) // (d // H)
               == lax.broadcasted_iota(jnp.int32, (H, d), 0)
               ).astype(jnp.float32)
        scale = lax.dot_general(sinv, rep, (((1,), (0,)), ((), ())),
                                preferred_element_type=jnp.float32)
        ybig = jnp.concatenate([planes[h][:, :d] for h in range(H)],
                               axis=1)
        x = jnp.dot(ybig, wvbig_ref[...],
                    preferred_element_type=jnp.float32) * scale

        def ln(v, g, b):
            mu = jnp.mean(v, axis=1, keepdims=True)
            var = jnp.mean(jnp.square(v - mu), axis=1, keepdims=True)
            return g * (v - mu) / jnp.sqrt(var + 1e-3) + b

        x = ln(x, g1_ref[...], b1_ref[...])
        x = jnp.maximum(jnp.dot(x, wo_ref[...],
                                preferred_element_type=jnp.float32)
                        + bo_ref[...], 0.0)
        x = ln(x, g2_ref[...], b2_ref[...])
        out_ref[...] = x + x0


def kernel(nf, nId, vnt, WQ, WK, WV, WO, bO, g1, b1, g2, b2):
    n, d = nf.shape
    dk = d // H
    seg = nId.astype(jnp.int32)

    q = (vnt @ WQ).reshape(H, dk)
    watt = (WK.reshape(d, H, dk) * q[None, :, :]).sum(-1) / math.sqrt(dk)
    wvr = WV.reshape(d, H, dk).transpose(1, 0, 2)
    wvbig = (jnp.eye(H, dtype=jnp.float32)[:, None, :, None]
             * wvr[:, :, None, :]).reshape(H * d, d)

    bsz = 1000
    assert n % bsz == 0
    nb = n // bsz
    seg3 = seg.reshape(nb, 1, bsz)

    out = pl.pallas_call(
        functools.partial(_fused_body, bsz=bsz, nb=nb, d=d),
        grid=(nb,),
        in_specs=[
            pl.BlockSpec((bsz, d), lambda i: (i, 0)),
            pl.BlockSpec((1, 1, bsz), lambda i: (i, 0, 0)),
            pl.BlockSpec((d, H), lambda i: (0, 0)),
            pl.BlockSpec((H * d, d), lambda i: (0, 0)),
            pl.BlockSpec((d, d), lambda i: (0, 0)),
            pl.BlockSpec((1, d), lambda i: (0, 0)),
            pl.BlockSpec((1, d), lambda i: (0, 0)),
            pl.BlockSpec((1, d), lambda i: (0, 0)),
            pl.BlockSpec((1, d), lambda i: (0, 0)),
            pl.BlockSpec((1, d), lambda i: (0, 0)),
        ],
        out_specs=pl.BlockSpec((G, d), lambda i: (0, 0)),
        out_shape=jax.ShapeDtypeStruct((G, d), jnp.float32),
        scratch_shapes=[pltpu.VMEM((GP, 16, AW), jnp.float32)],
    )(nf, seg3, watt, wvbig, WO, bO.reshape(1, d), g1.reshape(1, d),
      b1.reshape(1, d), g2.reshape(1, d), b2.reshape(1, d))
    return out


# speedup vs baseline: 1.2350x; 1.0023x over previous
import functools
import math

import jax
import jax.numpy as jnp
from jax import lax
from jax.experimental import pallas as pl
from jax.experimental.pallas import tpu as pltpu

G = 512
H = 8
SW = 16
GP = G + 2 * SW
AW = 384


def _fused_body(nf_ref, seg_ref, watt_ref, wvbig_ref, wo_ref, bo_ref,
                g1_ref, b1_ref, g2_ref, b2_ref, out_ref, acc_ref, *,
                bsz, nb, d):
    i = pl.program_id(0)

    @pl.when(i == 0)
    def _init():
        acc_ref[...] = jnp.zeros_like(acc_ref)

    nfb = nf_ref[...]
    aug = jnp.concatenate(
        [nfb, jnp.ones((bsz, 1), jnp.float32),
         jnp.zeros((bsz, AW - d - 1), jnp.float32)], axis=1)
    ltt = lax.dot_general(watt_ref[...], nfb, (((0,), (1,)), ((), ())),
                          preferred_element_type=jnp.float32)
    ett = jnp.exp(ltt)
    colj = lax.broadcasted_iota(jnp.int32, (H, 16 * SW), 1) % 16
    rowj = lax.broadcasted_iota(jnp.int32, (H, 16 * SW), 0)
    tilemat = (colj == rowj).astype(jnp.float32)
    const1 = (lax.broadcasted_iota(jnp.int32, (1, 16 * SW), 1) % 16
              == H).astype(jnp.float32)
    wtile = lax.dot_general(ett, tilemat, (((0,), (0,)), ((), ())),
                            preferred_element_type=jnp.float32) + const1
    segb2 = seg_ref[0, 0, :][:, None]
    colgrp = lax.broadcasted_iota(jnp.int32, (1, 16 * SW), 1) // 16
    lo = seg_ref[0, 0, 0]
    hi = seg_ref[0, 0, bsz - 1]
    nwin = (hi - lo) // SW + 1

    def win_body(jw, carry):
        base = lo + jw * SW
        match = (segb2 == base + colgrp)
        ew = jnp.where(match, wtile, 0.0)
        contrib = lax.dot_general(ew, aug, (((0,), (0,)), ((), ())),
                                  preferred_element_type=jnp.float32)
        acc_ref[pl.ds(base, SW), :, :] += contrib.reshape(SW, 16, AW)
        return carry

    lax.fori_loop(0, nwin, win_body, 0)

    @pl.when(i == nb - 1)
    def _epilogue():
        planes = [acc_ref[:G, h, :] for h in range(H + 1)]
        x0 = planes[H][:, :d]
        cinv = 1.0 / jnp.maximum(planes[H][:, d:d + 1], 1.0)
        sinv = jnp.concatenate([planes[h][:, d:d + 1] for h in range(H)],
                               axis=1)
        sinv = cinv / (sinv + 1e-16)
        rep = (lax.broadcasted_iota(jnp.int32, (H, d), 1) // (d // H)
               == lax.broadcasted_iota(jnp.int32, (H, d), 0)
               ).astype(jnp.float32)
        scale = lax.dot_general(sinv, rep, (((1,), (0,)), ((), ())),
                                preferred_element_type=jnp.float32)
        ybig = jnp.concatenate([planes[h][:, :d] for h in range(H)],
                               axis=1)
        x = jnp.dot(ybig, wvbig_ref[...],
                    preferred_element_type=jnp.float32) * scale

        def ln(v, g, b):
            mu = jnp.mean(v, axis=1, keepdims=True)
            var = jnp.mean(jnp.square(v - mu), axis=1, keepdims=True)
            return g * (v - mu) / jnp.sqrt(var + 1e-3) + b

        x = ln(x, g1_ref[...], b1_ref[...])
        x = jnp.maximum(jnp.dot(x, wo_ref[...],
                                preferred_element_type=jnp.float32)
                        + bo_ref[...], 0.0)
        x = ln(x, g2_ref[...], b2_ref[...])
        out_ref[...] = x + x0


def kernel(nf, nId, vnt, WQ, WK, WV, WO, bO, g1, b1, g2, b2):
    n, d = nf.shape
    dk = d // H
    seg = nId.astype(jnp.int32)

    q = (vnt @ WQ).reshape(H, dk)
    watt = (WK.reshape(d, H, dk) * q[None, :, :]).sum(-1) / math.sqrt(dk)
    wvr = WV.reshape(d, H, dk).transpose(1, 0, 2)
    wvbig = (jnp.eye(H, dtype=jnp.float32)[:, None, :, None]
             * wvr[:, :, None, :]).reshape(H * d, d)

    bsz = 2000
    assert n % bsz == 0
    nb = n // bsz
    seg3 = seg.reshape(nb, 1, bsz)

    out = pl.pallas_call(
        functools.partial(_fused_body, bsz=bsz, nb=nb, d=d),
        grid=(nb,),
        in_specs=[
            pl.BlockSpec((bsz, d), lambda i: (i, 0)),
            pl.BlockSpec((1, 1, bsz), lambda i: (i, 0, 0)),
            pl.BlockSpec((d, H), lambda i: (0, 0)),
            pl.BlockSpec((H * d, d), lambda i: (0, 0)),
            pl.BlockSpec((d, d), lambda i: (0, 0)),
            pl.BlockSpec((1, d), lambda i: (0, 0)),
            pl.BlockSpec((1, d), lambda i: (0, 0)),
            pl.BlockSpec((1, d), lambda i: (0, 0)),
            pl.BlockSpec((1, d), lambda i: (0, 0)),
            pl.BlockSpec((1, d), lambda i: (0, 0)),
        ],
        out_specs=pl.BlockSpec((G, d), lambda i: (0, 0)),
        out_shape=jax.ShapeDtypeStruct((G, d), jnp.float32),
        scratch_shapes=[pltpu.VMEM((GP, 16, AW), jnp.float32)],
    )(nf, seg3, watt, wvbig, WO, bO.reshape(1, d), g1.reshape(1, d),
      b1.reshape(1, d), g2.reshape(1, d), b2.reshape(1, d))
    return out


# speedup vs baseline: 1.2372x; 1.0018x over previous
import functools
import math

import jax
import jax.numpy as jnp
from jax import lax
from jax.experimental import pallas as pl
from jax.experimental.pallas import tpu as pltpu

G = 512
H = 8
SW = 16
GP = G + 2 * SW
AW = 384


def _fused_body(nf_ref, seg_ref, watt_ref, wvbig_ref, wo_ref, bo_ref,
                g1_ref, b1_ref, g2_ref, b2_ref, out_ref, acc_ref, *,
                bsz, nb, d):
    i = pl.program_id(0)

    @pl.when(i == 0)
    def _init():
        acc_ref[...] = jnp.zeros_like(acc_ref)

    nfb = nf_ref[...]
    aug = jnp.concatenate(
        [nfb, jnp.ones((bsz, 1), jnp.float32),
         jnp.zeros((bsz, AW - d - 1), jnp.float32)], axis=1)
    ltt = lax.dot_general(watt_ref[...], nfb, (((0,), (1,)), ((), ())),
                          preferred_element_type=jnp.float32)
    ett = jnp.exp(ltt)
    colj = lax.broadcasted_iota(jnp.int32, (H, 16 * SW), 1) % 16
    rowj = lax.broadcasted_iota(jnp.int32, (H, 16 * SW), 0)
    tilemat = (colj == rowj).astype(jnp.float32)
    const1 = (lax.broadcasted_iota(jnp.int32, (1, 16 * SW), 1) % 16
              == H).astype(jnp.float32)
    wtile = lax.dot_general(ett, tilemat, (((0,), (0,)), ((), ())),
                            preferred_element_type=jnp.float32) + const1
    wtileb = wtile.astype(jnp.bfloat16)
    augb = aug.astype(jnp.bfloat16)
    segb2 = seg_ref[0, 0, :][:, None]
    colgrp = lax.broadcasted_iota(jnp.int32, (1, 16 * SW), 1) // 16
    lo = seg_ref[0, 0, 0]
    hi = seg_ref[0, 0, bsz - 1]
    nwin = (hi - lo) // SW + 1

    def win_body(jw, carry):
        base = lo + jw * SW
        match = (segb2 == base + colgrp)
        ew = jnp.where(match, wtileb, jnp.bfloat16(0.0))
        contrib = lax.dot_general(ew, augb, (((0,), (0,)), ((), ())),
                                  preferred_element_type=jnp.float32)
        acc_ref[pl.ds(base, SW), :, :] += contrib.reshape(SW, 16, AW)
        return carry

    lax.fori_loop(0, nwin, win_body, 0)

    @pl.when(i == nb - 1)
    def _epilogue():
        planes = [acc_ref[:G, h, :] for h in range(H + 1)]
        x0 = planes[H][:, :d]
        cinv = 1.0 / jnp.maximum(planes[H][:, d:d + 1], 1.0)
        sinv = jnp.concatenate([planes[h][:, d:d + 1] for h in range(H)],
                               axis=1)
        sinv = cinv / (sinv + 1e-16)
        rep = (lax.broadcasted_iota(jnp.int32, (H, d), 1) // (d // H)
               == lax.broadcasted_iota(jnp.int32, (H, d), 0)
               ).astype(jnp.float32)
        scale = lax.dot_general(sinv, rep, (((1,), (0,)), ((), ())),
                                preferred_element_type=jnp.float32)
        ybig = jnp.concatenate([planes[h][:, :d] for h in range(H)],
                               axis=1)
        x = jnp.dot(ybig, wvbig_ref[...],
                    preferred_element_type=jnp.float32) * scale

        def ln(v, g, b):
            mu = jnp.mean(v, axis=1, keepdims=True)
            var = jnp.mean(jnp.square(v - mu), axis=1, keepdims=True)
            return g * (v - mu) / jnp.sqrt(var + 1e-3) + b

        x = ln(x, g1_ref[...], b1_ref[...])
        x = jnp.maximum(jnp.dot(x, wo_ref[...],
                                preferred_element_type=jnp.float32)
                        + bo_ref[...], 0.0)
        x = ln(x, g2_ref[...], b2_ref[...])
        out_ref[...] = x + x0


def kernel(nf, nId, vnt, WQ, WK, WV, WO, bO, g1, b1, g2, b2):
    n, d = nf.shape
    dk = d // H
    seg = nId.astype(jnp.int32)

    q = (vnt @ WQ).reshape(H, dk)
    watt = (WK.reshape(d, H, dk) * q[None, :, :]).sum(-1) / math.sqrt(dk)
    wvr = WV.reshape(d, H, dk).transpose(1, 0, 2)
    wvbig = (jnp.eye(H, dtype=jnp.float32)[:, None, :, None]
             * wvr[:, :, None, :]).reshape(H * d, d)

    bsz = 1000
    assert n % bsz == 0
    nb = n // bsz
    seg3 = seg.reshape(nb, 1, bsz)

    out = pl.pallas_call(
        functools.partial(_fused_body, bsz=bsz, nb=nb, d=d),
        grid=(nb,),
        in_specs=[
            pl.BlockSpec((bsz, d), lambda i: (i, 0)),
            pl.BlockSpec((1, 1, bsz), lambda i: (i, 0, 0)),
            pl.BlockSpec((d, H), lambda i: (0, 0)),
            pl.BlockSpec((H * d, d), lambda i: (0, 0)),
            pl.BlockSpec((d, d), lambda i: (0, 0)),
            pl.BlockSpec((1, d), lambda i: (0, 0)),
            pl.BlockSpec((1, d), lambda i: (0, 0)),
            pl.BlockSpec((1, d), lambda i: (0, 0)),
            pl.BlockSpec((1, d), lambda i: (0, 0)),
            pl.BlockSpec((1, d), lambda i: (0, 0)),
        ],
        out_specs=pl.BlockSpec((G, d), lambda i: (0, 0)),
        out_shape=jax.ShapeDtypeStruct((G, d), jnp.float32),
        scratch_shapes=[pltpu.VMEM((GP, 16, AW), jnp.float32)],
    )(nf, seg3, watt, wvbig, WO, bO.reshape(1, d), g1.reshape(1, d),
      b1.reshape(1, d), g2.reshape(1, d), b2.reshape(1, d))
    return out
